# baseline (speedup 1.0000x reference)
import functools

import jax
import jax.numpy as jnp
from jax import lax
from jax.experimental import pallas as pl
from jax.experimental.pallas import tpu as pltpu
from jax.experimental.pallas import tpu_sc as plsc

B, C, H, W = 2, 64, 184, 180
HW = H * W
D = B * C
DW = D // 2
NPTS = 16384
NV = 32

_info = plsc.get_sparse_core_info()
NC, NS, L = _info.num_cores, _info.num_subcores, _info.num_lanes
NW = NC * NS
PW = NPTS // NW
NPC = 4
NCHUNK = PW // NPC
IDX_PER_CHUNK = NPC * NV
WG = DW // L

CPT = 1040
HW_PAD = NW * CPT
HALves = (528, 512)

_params = pltpu.CompilerParams(
    needs_layout_passes=False, use_tc_tiling_on_sc=False)
_mesh = plsc.VectorSubcoreMesh(core_axis_name="c", subcore_axis_name="s")


def _sc_transpose(words):
    @functools.partial(
        pl.kernel,
        mesh=_mesh,
        out_type=jax.ShapeDtypeStruct((HW_PAD, DW), jnp.int32),
        compiler_params=_params,
        scratch_types=[
            pltpu.VMEM((DW, HALves[0]), jnp.int32),
            pltpu.VMEM((HALves[0], DW), jnp.int32),
        ],
    )
    def k(words_hbm, table_hbm, in_v, out_v):
        wid = lax.axis_index("s") * NC + lax.axis_index("c")
        base = wid * CPT
        row16 = lax.iota(jnp.int32, L)
        off = 0
        for half in HALves:
            cell0 = base + off
            pltpu.sync_copy(words_hbm.at[:, pl.ds(cell0, half)],
                            in_v.at[:, pl.ds(0, half)])
            for c in range(DW):
                col = jnp.full((L,), c, jnp.int32)

                def g_body(g, _, c=c, col=col, half=half):
                    w = in_v[c, pl.ds(g * L, L)]
                    plsc.store_scatter(out_v, [g * L + row16, col], w)
                    return 0

                lax.fori_loop(0, half // L, g_body, 0)
            pltpu.sync_copy(out_v.at[pl.ds(0, half)],
                            table_hbm.at[pl.ds(cell0, half)])
            off += half

    return k(words)


def _sc_gather_mean(table, idx):
    @functools.partial(
        pl.kernel,
        mesh=_mesh,
        out_type=jax.ShapeDtypeStruct((NPTS, D), jnp.float32),
        compiler_params=_params,
        scratch_types=[
            pltpu.VMEM((NCHUNK, IDX_PER_CHUNK), jnp.int32),
            pltpu.VMEM((2, IDX_PER_CHUNK, DW), jnp.int32),
            pltpu.VMEM((PW, D), jnp.float32),
            pltpu.SemaphoreType.DMA,
            pltpu.SemaphoreType.DMA,
        ],
    )
    def k(table_hbm, idx_hbm, out_hbm, idx_v, rows_v, outst_v, sem0, sem1):
        wid = lax.axis_index("s") * NC + lax.axis_index("c")
        pltpu.sync_copy(idx_hbm.at[wid], idx_v)
        sems = (sem0, sem1)
        himask = jnp.full((L,), -65536, jnp.int32)

        def gather(ci, b, sem):
            return pltpu.make_async_copy(
                table_hbm.at[idx_v.at[ci]], rows_v.at[b], sem)

        gather(0, 0, sem0).start()
        gather(1, 1, sem1).start()

        def pair_body(g, _):
            for b in range(2):
                ci = g * 2 + b
                gather(ci, b, sems[b]).wait()

                def pt_body(j, _):
                    base = j * NV
                    row = ci * NPC + j
                    for wg in range(WG):
                        col = wg * L
                        w = rows_v[b, base, pl.ds(col, L)]
                        lo = plsc.bitcast(lax.shift_left(w, 16), jnp.float32)
                        hi = plsc.bitcast(lax.bitwise_and(w, himask), jnp.float32)
                        for r in range(1, NV):
                            w = rows_v[b, base + r, pl.ds(col, L)]
                            lo = lo + plsc.bitcast(
                                lax.shift_left(w, 16), jnp.float32)
                            hi = hi + plsc.bitcast(
                                lax.bitwise_and(w, himask), jnp.float32)
                        outst_v[row, pl.ds(wg * L, L)] = lo
                        outst_v[row, pl.ds(DW + wg * L, L)] = hi
                    return 0

                lax.fori_loop(0, NPC, pt_body, 0)

                @pl.when(ci + 2 < NCHUNK)
                def _():
                    gather(ci + 2, b, sems[b]).start()

            return 0

        lax.fori_loop(0, NCHUNK // 2, pair_body, 0)
        pltpu.sync_copy(outst_v, out_hbm.at[pl.ds(wid * PW, PW)])

    return k(table, idx)


def kernel(feats, mapping):
    scaled = ((feats * (1.0 / NV)).astype(jnp.bfloat16)
              .reshape(B, C, HW))
    u = jax.lax.bitcast_convert_type(scaled, jnp.uint16)
    words = u[0].astype(jnp.uint32) | (u[1].astype(jnp.uint32) << 16)
    words = jnp.pad(jax.lax.bitcast_convert_type(words, jnp.int32),
                    ((0, 0), (0, HW_PAD - HW)))
    tab_i32 = _sc_transpose(words)
    idx = mapping.reshape(NW, NCHUNK, IDX_PER_CHUNK)
    out_rows = _sc_gather_mean(tab_i32, idx)
    return jnp.transpose(out_rows).reshape(B, C, NPTS, 1)


# speedup vs baseline: 1.5906x; 1.5906x over previous
import functools

import jax
import jax.numpy as jnp
from jax import lax
from jax.experimental import pallas as pl
from jax.experimental.pallas import tpu as pltpu
from jax.experimental.pallas import tpu_sc as plsc

B, C, H, W = 2, 64, 184, 180
HW = H * W
D = B * C
DW = D // 2
NPTS = 16384
NV = 32

_info = plsc.get_sparse_core_info()
NC, NS, L = _info.num_cores, _info.num_subcores, _info.num_lanes
NW = NC * NS
PW = NPTS // NW
NPC = 4
NCHUNK = PW // NPC
IDX_PER_CHUNK = NPC * NV
WG = DW // L


def _sc_gather_mean(table, idx):
    mesh = plsc.VectorSubcoreMesh(core_axis_name="c", subcore_axis_name="s")

    @functools.partial(
        pl.kernel,
        mesh=mesh,
        out_type=jax.ShapeDtypeStruct((D, NPTS), jnp.float32),
        compiler_params=pltpu.CompilerParams(
            needs_layout_passes=False, use_tc_tiling_on_sc=False),
        scratch_types=[
            pltpu.VMEM((NCHUNK * IDX_PER_CHUNK,), jnp.int32),
            pltpu.VMEM((2, IDX_PER_CHUNK, DW), jnp.int32),
            pltpu.VMEM((D, PW), jnp.float32),
            pltpu.SemaphoreType.DMA,
            pltpu.SemaphoreType.DMA,
        ],
    )
    def k(table_hbm, idx_hbm, out_hbm, idx_v, rows_v, outst_v, sem0, sem1):
        wid = lax.axis_index("s") * NC + lax.axis_index("c")
        pltpu.sync_copy(
            idx_hbm.at[pl.ds(wid * (PW * NV), PW * NV)], idx_v)
        sems = (sem0, sem1)
        himask = jnp.full((L,), -65536, jnp.int32)
        row16 = lax.iota(jnp.int32, L)
        lo_rows = [wg * L + row16 for wg in range(WG)]
        hi_rows = [DW + wg * L + row16 for wg in range(WG)]

        def gather(ci, b, sem):
            return pltpu.make_async_copy(
                table_hbm.at[idx_v.at[pl.ds(ci * IDX_PER_CHUNK, IDX_PER_CHUNK)]],
                rows_v.at[b], sem)

        gather(0, 0, sem0).start()
        gather(1, 1, sem1).start()

        def pair_body(g, _):
            for b in range(2):
                ci = g * 2 + b
                gather(ci, b, sems[b]).wait()

                def pt_body(j, _):
                    base = j * NV
                    pcol = jnp.full((L,), ci * NPC + j, jnp.int32)
                    for wg in range(WG):
                        col = wg * L
                        w = rows_v[b, base, pl.ds(col, L)]
                        lo = plsc.bitcast(lax.shift_left(w, 16), jnp.float32)
                        hi = plsc.bitcast(lax.bitwise_and(w, himask), jnp.float32)
                        for r in range(1, NV):
                            w = rows_v[b, base + r, pl.ds(col, L)]
                            lo = lo + plsc.bitcast(
                                lax.shift_left(w, 16), jnp.float32)
                            hi = hi + plsc.bitcast(
                                lax.bitwise_and(w, himask), jnp.float32)
                        plsc.store_scatter(outst_v, [lo_rows[wg], pcol], lo)
                        plsc.store_scatter(outst_v, [hi_rows[wg], pcol], hi)
                    return 0

                lax.fori_loop(0, NPC, pt_body, 0)

                @pl.when(ci + 2 < NCHUNK)
                def _():
                    gather(ci + 2, b, sems[b]).start()

            return 0

        lax.fori_loop(0, NCHUNK // 2, pair_body, 0)
        pltpu.sync_copy(outst_v, out_hbm.at[:, pl.ds(wid * PW, PW)])

    return k(table, idx)


def kernel(feats, mapping):
    scaled = ((feats * (1.0 / NV)).astype(jnp.bfloat16)
              .reshape(B, C, HW))
    u = jax.lax.bitcast_convert_type(scaled, jnp.uint16)
    words = u[0].astype(jnp.uint32) | (u[1].astype(jnp.uint32) << 16)
    tab_i32 = jnp.transpose(
        jax.lax.bitcast_convert_type(words, jnp.int32))
    out_t = _sc_gather_mean(tab_i32, mapping.reshape(-1))
    return out_t.reshape(B, C, NPTS, 1)
